# baseline (speedup 1.0000x reference)
import functools

import jax
import jax.numpy as jnp
from jax import lax
from jax.experimental import pallas as pl
from jax.experimental.pallas import tpu as pltpu
from jax.experimental.pallas import tpu_sc as plsc

_SCALE = 10.0
_KTOP = 10
_TEMP = 0.5
_NEG_INF = -1e30
_GONE = -3.0e38


def _l2n(x):
    n = jnp.sqrt(jnp.sum(x * x, axis=-1, keepdims=True))
    return x / jnp.maximum(n, 1e-12)


def _bdot(a, b):
    return lax.dot_general(a.astype(jnp.bfloat16), b.astype(jnp.bfloat16),
                           (((1,), (1,)), ((), ())),
                           preferred_element_type=jnp.float32)


def _k1_body(gif_ref, gtf_ref, ltf_ref, mem_ref, sim_ref, bm_ref, align_ref,
             tm_ref):
    m = pl.program_id(0)

    @pl.when(m == 0)
    def _():
        B = gif_ref.shape[0]
        img = _l2n(gif_ref[...])
        txt = _l2n(gtf_ref[...])
        diag = (lax.broadcasted_iota(jnp.int32, (B, B), 0)
                == lax.broadcasted_iota(jnp.int32, (B, B), 1))

        def sim_logits(x):
            return jnp.where(diag, _NEG_INF, _bdot(x, x) / _TEMP)

        img_sim = sim_logits(img)
        txt_sim = sim_logits(txt)

        def log_softmax(s):
            z = s - jnp.max(s, axis=1, keepdims=True)
            return z - jnp.log(jnp.sum(jnp.exp(z), axis=1, keepdims=True))

        img_logp = log_softmax(img_sim)
        txt_logp = log_softmax(txt_sim)
        img_p = jnp.exp(img_logp)
        txt_p = jnp.exp(txt_logp)

        def kl(logp, p):
            return jnp.sum(p * (jnp.log(jnp.maximum(p, 1e-12)) - logp)) / B

        align_ref[...] = jnp.reshape(
            0.5 * (kl(img_logp, txt_p) + kl(txt_logp, img_p)), (1, 1))
        tm_ref[...] = _l2n(jnp.mean(ltf_ref[...], axis=0))

    s = _bdot(tm_ref[...], _l2n(mem_ref[...]))
    sim_ref[...] = s
    B, mb1 = s.shape
    bm_ref[...] = jnp.max(s.reshape(B, mb1 // 128, 128), axis=2)[None]


def _sc_topk_body(sim_hbm, bm_hbm, vid_hbm, idx_hbm, pv_hbm,
                  simv, bmv, vidv, idxv, pvv):
    nc = 2
    wid = lax.axis_index("s") * nc + lax.axis_index("c")
    base = wid * 4
    pltpu.sync_copy(sim_hbm.at[pl.ds(base, 4)], simv)
    pltpu.sync_copy(bm_hbm.at[pl.ds(base, 4)], bmv)
    pltpu.sync_copy(vid_hbm, vidv)

    lane = lax.broadcasted_iota(jnp.int32, (16,), 0)

    idxvecs = [jnp.zeros((16,), jnp.int32) for _ in range(4)]
    for k in range(_KTOP):
        for r in range(4):
            b0 = bmv[r, pl.ds(0, 16)]
            b1 = bmv[r, pl.ds(16, 16)]
            b2 = bmv[r, pl.ds(32, 16)]
            b3 = bmv[r, pl.ds(48, 16)]
            gmax = jnp.max(jnp.maximum(jnp.maximum(b0, b1),
                                       jnp.maximum(b2, b3)))
            cpos = jnp.full((16,), 64, jnp.int32)
            for i, bi in enumerate((b0, b1, b2, b3)):
                cpos = jnp.minimum(cpos,
                                   jnp.where(bi == gmax, lane + 16 * i, 64))
            cstar = jnp.min(cpos)
            cbase = cstar * 128
            tpos = jnp.full((16,), 128, jnp.int32)
            for j in range(8):
                vj = simv[r, pl.ds(cbase + 16 * j, 16)]
                tpos = jnp.minimum(tpos,
                                   jnp.where(vj == gmax, lane + 16 * j, 128))
            mstar = cbase + jnp.min(tpos)
            idxvecs[r] = jnp.where(lane == k, mstar, idxvecs[r])
            plsc.store_scatter(simv,
                               [jnp.full((16,), r, jnp.int32),
                                jnp.full((16,), mstar, jnp.int32)],
                               jnp.full((16,), _GONE, jnp.float32),
                               mask=lane == 0)
            newmax = jnp.full((16,), _GONE, jnp.float32)
            for j in range(8):
                newmax = jnp.maximum(newmax,
                                     simv[r, pl.ds(cbase + 16 * j, 16)])
            plsc.store_scatter(bmv,
                               [jnp.full((16,), r, jnp.int32),
                                jnp.full((16,), cstar, jnp.int32)],
                               jnp.full((16,), jnp.max(newmax), jnp.float32),
                               mask=lane == 0)
    for r in range(4):
        idxv[r, :] = idxvecs[r]
        pvv[r, :] = plsc.load_gather(vidv, [idxvecs[r]])

    pltpu.sync_copy(idxv, idx_hbm.at[pl.ds(base, 4)])
    pltpu.sync_copy(pvv, pv_hbm.at[pl.ds(base, 4)])


def _k3_body(lif_ref, cen_ref, idx_ref, pos_ref, align_ref,
             total_ref, local_ref, spos_ref, sneg_ref, h_ref, msk_ref):
    mb = pl.program_id(0)
    p = pl.program_id(1)
    num_mb = pl.num_programs(0)
    num_p = pl.num_programs(1)

    pf = lif_ref[0]
    cb = cen_ref[0]
    B = pf.shape[0]
    blk = cb.shape[0]

    @pl.when(jnp.logical_and(mb == 0, p == 0))
    def _():
        spos_ref[...] = jnp.zeros_like(spos_ref)
        sneg_ref[...] = jnp.zeros_like(sneg_ref)

    @pl.when(p == 0)
    def _():
        base = mb * blk
        col = base + lax.broadcasted_iota(jnp.int32, (B, blk), 1)
        h = jnp.zeros((B, blk), jnp.float32)
        for k in range(_KTOP):
            h = h + (col == idx_ref[:, k:k + 1]).astype(jnp.float32)
        h_ref[...] = h
        hit = (jnp.sum(h, axis=0, keepdims=True)
               + jnp.sum((col == pos_ref[...]).astype(jnp.float32), axis=0,
                         keepdims=True))
        msk_ref[...] = jnp.where(hit > 0.0, 0.0, 1.0)

    a2 = jnp.sum(pf * pf, axis=1, keepdims=True)
    c2 = jnp.sum(cb * cb, axis=1, keepdims=True)
    d2 = a2 + c2.T - 2.0 * _bdot(pf, cb)
    dist = jnp.sqrt(jnp.maximum(d2, 1e-12))
    e = jnp.exp(-_SCALE * dist)

    lanes = pl.ds(p * B, B)
    spos_ref[:, lanes] += jnp.sum(e * h_ref[...], axis=1, keepdims=True).T
    sneg_ref[:, lanes] += jnp.sum(e * msk_ref[...], axis=1, keepdims=True).T

    @pl.when(jnp.logical_and(mb == num_mb - 1, p == num_p - 1))
    def _():
        acc = jnp.zeros((1, 1), jnp.float32)
        for q in range(4):
            x = jnp.log(spos_ref[:, q * B:(q + 1) * B])
            y = jnp.log(sneg_ref[:, q * B:(q + 1) * B])
            l = jnp.sum(y - x) / B
            l = jnp.where(jnp.isnan(l), 0.0, l)
            acc = acc + jnp.reshape(l, (1, 1))
        loc = acc / num_p
        local_ref[...] = loc
        total_ref[...] = loc + 0.5 * align_ref[...]


def _impl(global_img_feat, global_text_feat, local_img_feats,
          local_text_feats, centers, text_centers, memory_feats,
          position, memory_vid, interpret=False):
    B, d = global_img_feat.shape
    P, M, _ = centers.shape
    pos2d = position.reshape(B, 1)

    nb1 = 8
    mb1 = M // nb1
    sim, bm, align = pl.pallas_call(
        _k1_body,
        grid=(nb1,),
        in_specs=[
            pl.BlockSpec((B, d), lambda m: (0, 0)),
            pl.BlockSpec((B, d), lambda m: (0, 0)),
            pl.BlockSpec((P, B, d), lambda m: (0, 0, 0)),
            pl.BlockSpec((mb1, d), lambda m: (m, 0)),
        ],
        out_specs=[
            pl.BlockSpec((B, mb1), lambda m: (0, m)),
            pl.BlockSpec((1, B, mb1 // 128), lambda m: (m, 0, 0)),
            pl.BlockSpec((1, 1), lambda m: (0, 0)),
        ],
        out_shape=[jax.ShapeDtypeStruct((B, M), jnp.float32),
                   jax.ShapeDtypeStruct((nb1, B, mb1 // 128), jnp.float32),
                   jax.ShapeDtypeStruct((1, 1), jnp.float32)],
        scratch_shapes=[pltpu.VMEM((B, d), jnp.float32)],
        interpret=interpret,
    )(global_img_feat, global_text_feat, local_text_feats, memory_feats)
    bm = bm.transpose(1, 0, 2).reshape(B, M // 128)

    sc_topk = functools.partial(
        pl.kernel,
        out_type=[jax.ShapeDtypeStruct((B, 16), jnp.int32),
                  jax.ShapeDtypeStruct((B, 16), jnp.int32)],
        mesh=plsc.VectorSubcoreMesh(core_axis_name="c", subcore_axis_name="s"),
        compiler_params=pltpu.CompilerParams(needs_layout_passes=False),
        scratch_types=[pltpu.VMEM((4, M), jnp.float32),
                       pltpu.VMEM((4, M // 128), jnp.float32),
                       pltpu.VMEM((M,), jnp.int32),
                       pltpu.VMEM((4, 16), jnp.int32),
                       pltpu.VMEM((4, 16), jnp.int32)],
        interpret=interpret,
    )(_sc_topk_body)
    idx, pv = sc_topk(sim, bm, memory_vid)

    nblk = 8
    blk = M // nblk
    total, local = pl.pallas_call(
        _k3_body,
        grid=(nblk, P),
        in_specs=[
            pl.BlockSpec((1, B, d), lambda m, p: (p, 0, 0)),
            pl.BlockSpec((1, blk, d), lambda m, p: (p, m, 0)),
            pl.BlockSpec((B, 16), lambda m, p: (0, 0)),
            pl.BlockSpec((B, 1), lambda m, p: (0, 0)),
            pl.BlockSpec((1, 1), lambda m, p: (0, 0)),
        ],
        out_specs=[
            pl.BlockSpec((1, 1), lambda m, p: (0, 0)),
            pl.BlockSpec((1, 1), lambda m, p: (0, 0)),
        ],
        out_shape=[jax.ShapeDtypeStruct((1, 1), jnp.float32),
                   jax.ShapeDtypeStruct((1, 1), jnp.float32)],
        scratch_shapes=[pltpu.VMEM((1, P * B), jnp.float32),
                        pltpu.VMEM((1, P * B), jnp.float32),
                        pltpu.VMEM((B, blk), jnp.float32),
                        pltpu.VMEM((1, blk), jnp.float32)],
        interpret=interpret,
    )(local_img_feats, centers, idx, pos2d, align)

    return (total[0, 0], local[0, 0], align[0, 0], pv[:, :_KTOP])


def kernel(global_img_feat, global_text_feat, local_img_feats,
           local_text_feats, centers, text_centers, memory_feats,
           position, memory_vid):
    return _impl(global_img_feat, global_text_feat, local_img_feats,
                 local_text_feats, centers, text_centers, memory_feats,
                 position, memory_vid)


# speedup vs baseline: 1.0025x; 1.0025x over previous
import functools

import jax
import jax.numpy as jnp
from jax import lax
from jax.experimental import pallas as pl
from jax.experimental.pallas import tpu as pltpu
from jax.experimental.pallas import tpu_sc as plsc

_SCALE = 10.0
_KTOP = 10
_TEMP = 0.5
_NEG_INF = -1e30
_GONE = -3.0e38


def _l2n(x):
    n = jnp.sqrt(jnp.sum(x * x, axis=-1, keepdims=True))
    return x / jnp.maximum(n, 1e-12)


def _bdot(a, b):
    return lax.dot_general(a.astype(jnp.bfloat16), b.astype(jnp.bfloat16),
                           (((1,), (1,)), ((), ())),
                           preferred_element_type=jnp.float32)


def _align_body(gif_ref, gtf_ref, align_ref):
    B = gif_ref.shape[0]
    img = _l2n(gif_ref[...])
    txt = _l2n(gtf_ref[...])
    diag = (lax.broadcasted_iota(jnp.int32, (B, B), 0)
            == lax.broadcasted_iota(jnp.int32, (B, B), 1))

    def sim_logits(x):
        return jnp.where(diag, _NEG_INF, _bdot(x, x) / _TEMP)

    img_sim = sim_logits(img)
    txt_sim = sim_logits(txt)

    def log_softmax(s):
        z = s - jnp.max(s, axis=1, keepdims=True)
        return z - jnp.log(jnp.sum(jnp.exp(z), axis=1, keepdims=True))

    img_logp = log_softmax(img_sim)
    txt_logp = log_softmax(txt_sim)
    img_p = jnp.exp(img_logp)
    txt_p = jnp.exp(txt_logp)

    def kl(logp, p):
        return jnp.sum(p * (jnp.log(jnp.maximum(p, 1e-12)) - logp)) / B

    align_ref[...] = jnp.reshape(
        0.5 * (kl(img_logp, txt_p) + kl(txt_logp, img_p)), (1, 1))


def _k1_body(ltf_ref, mem_ref, sim_ref, bm_ref, tm_ref):
    m = pl.program_id(0)

    @pl.when(m == 0)
    def _():
        tm_ref[...] = _l2n(jnp.mean(ltf_ref[...], axis=0))

    s = _bdot(tm_ref[...], _l2n(mem_ref[...]))
    sim_ref[...] = s
    B, mb1 = s.shape
    bm_ref[...] = jnp.max(s.reshape(B, mb1 // 128, 128), axis=2)[None]


def _sc_topk_body(sim_hbm, bm_hbm, vid_hbm, idx_hbm, pv_hbm,
                  simv, bmv, vidv, idxv, pvv):
    nc = 2
    wid = lax.axis_index("s") * nc + lax.axis_index("c")
    base = wid * 4
    pltpu.sync_copy(sim_hbm.at[pl.ds(base, 4)], simv)
    pltpu.sync_copy(bm_hbm.at[pl.ds(base, 4)], bmv)
    pltpu.sync_copy(vid_hbm, vidv)

    lane = lax.broadcasted_iota(jnp.int32, (16,), 0)

    idxvecs = [jnp.zeros((16,), jnp.int32) for _ in range(4)]
    for k in range(_KTOP):
        for r in range(4):
            b0 = bmv[r, pl.ds(0, 16)]
            b1 = bmv[r, pl.ds(16, 16)]
            b2 = bmv[r, pl.ds(32, 16)]
            b3 = bmv[r, pl.ds(48, 16)]
            gmax = jnp.max(jnp.maximum(jnp.maximum(b0, b1),
                                       jnp.maximum(b2, b3)))
            cpos = jnp.full((16,), 64, jnp.int32)
            for i, bi in enumerate((b0, b1, b2, b3)):
                cpos = jnp.minimum(cpos,
                                   jnp.where(bi == gmax, lane + 16 * i, 64))
            cstar = jnp.min(cpos)
            cbase = cstar * 128
            tpos = jnp.full((16,), 128, jnp.int32)
            for j in range(8):
                vj = simv[r, pl.ds(cbase + 16 * j, 16)]
                tpos = jnp.minimum(tpos,
                                   jnp.where(vj == gmax, lane + 16 * j, 128))
            mstar = cbase + jnp.min(tpos)
            idxvecs[r] = jnp.where(lane == k, mstar, idxvecs[r])
            plsc.store_scatter(simv,
                               [jnp.full((16,), r, jnp.int32),
                                jnp.full((16,), mstar, jnp.int32)],
                               jnp.full((16,), _GONE, jnp.float32),
                               mask=lane == 0)
            newmax = jnp.full((16,), _GONE, jnp.float32)
            for j in range(8):
                newmax = jnp.maximum(newmax,
                                     simv[r, pl.ds(cbase + 16 * j, 16)])
            plsc.store_scatter(bmv,
                               [jnp.full((16,), r, jnp.int32),
                                jnp.full((16,), cstar, jnp.int32)],
                               jnp.full((16,), jnp.max(newmax), jnp.float32),
                               mask=lane == 0)
    for r in range(4):
        idxv[r, :] = idxvecs[r]
        pvv[r, :] = plsc.load_gather(vidv, [idxvecs[r]])

    pltpu.sync_copy(idxv, idx_hbm.at[pl.ds(base, 4)])
    pltpu.sync_copy(pvv, pv_hbm.at[pl.ds(base, 4)])


def _k3_body(lif_ref, cen_ref, idx_ref, pos_ref, align_ref,
             total_ref, local_ref, spos_ref, sneg_ref, h_ref, msk_ref):
    mb = pl.program_id(0)
    p = pl.program_id(1)
    num_mb = pl.num_programs(0)
    num_p = pl.num_programs(1)

    pf = lif_ref[0]
    cb = cen_ref[0]
    B = pf.shape[0]
    blk = cb.shape[0]

    @pl.when(jnp.logical_and(mb == 0, p == 0))
    def _():
        spos_ref[...] = jnp.zeros_like(spos_ref)
        sneg_ref[...] = jnp.zeros_like(sneg_ref)

    @pl.when(p == 0)
    def _():
        base = mb * blk
        col = base + lax.broadcasted_iota(jnp.int32, (B, blk), 1)
        h = jnp.zeros((B, blk), jnp.float32)
        for k in range(_KTOP):
            h = h + (col == idx_ref[:, k:k + 1]).astype(jnp.float32)
        h_ref[...] = h
        hit = (jnp.sum(h, axis=0, keepdims=True)
               + jnp.sum((col == pos_ref[...]).astype(jnp.float32), axis=0,
                         keepdims=True))
        msk_ref[...] = jnp.where(hit > 0.0, 0.0, 1.0)

    a2 = jnp.sum(pf * pf, axis=1, keepdims=True)
    c2 = jnp.sum(cb * cb, axis=1, keepdims=True)
    d2 = a2 + c2.T - 2.0 * _bdot(pf, cb)
    dist = jnp.sqrt(jnp.maximum(d2, 1e-12))
    e = jnp.exp(-_SCALE * dist)

    lanes = pl.ds(p * B, B)
    spos_ref[:, lanes] += jnp.sum(e * h_ref[...], axis=1, keepdims=True).T
    sneg_ref[:, lanes] += jnp.sum(e * msk_ref[...], axis=1, keepdims=True).T

    @pl.when(jnp.logical_and(mb == num_mb - 1, p == num_p - 1))
    def _():
        acc = jnp.zeros((1, 1), jnp.float32)
        for q in range(4):
            x = jnp.log(spos_ref[:, q * B:(q + 1) * B])
            y = jnp.log(sneg_ref[:, q * B:(q + 1) * B])
            l = jnp.sum(y - x) / B
            l = jnp.where(jnp.isnan(l), 0.0, l)
            acc = acc + jnp.reshape(l, (1, 1))
        loc = acc / num_p
        local_ref[...] = loc
        total_ref[...] = loc + 0.5 * align_ref[...]


def _impl(global_img_feat, global_text_feat, local_img_feats,
          local_text_feats, centers, text_centers, memory_feats,
          position, memory_vid, interpret=False):
    B, d = global_img_feat.shape
    P, M, _ = centers.shape
    pos2d = position.reshape(B, 1)

    nb1 = 8
    mb1 = M // nb1
    sim, bm = pl.pallas_call(
        _k1_body,
        grid=(nb1,),
        in_specs=[
            pl.BlockSpec((P, B, d), lambda m: (0, 0, 0)),
            pl.BlockSpec((mb1, d), lambda m: (m, 0)),
        ],
        out_specs=[
            pl.BlockSpec((B, mb1), lambda m: (0, m)),
            pl.BlockSpec((1, B, mb1 // 128), lambda m: (m, 0, 0)),
        ],
        out_shape=[jax.ShapeDtypeStruct((B, M), jnp.float32),
                   jax.ShapeDtypeStruct((nb1, B, mb1 // 128), jnp.float32)],
        scratch_shapes=[pltpu.VMEM((B, d), jnp.float32)],
        interpret=interpret,
    )(local_text_feats, memory_feats)
    bm = bm.transpose(1, 0, 2).reshape(B, M // 128)

    align = pl.pallas_call(
        _align_body,
        out_shape=jax.ShapeDtypeStruct((1, 1), jnp.float32),
        interpret=interpret,
    )(global_img_feat, global_text_feat)

    sc_topk = functools.partial(
        pl.kernel,
        out_type=[jax.ShapeDtypeStruct((B, 16), jnp.int32),
                  jax.ShapeDtypeStruct((B, 16), jnp.int32)],
        mesh=plsc.VectorSubcoreMesh(core_axis_name="c", subcore_axis_name="s"),
        compiler_params=pltpu.CompilerParams(needs_layout_passes=False),
        scratch_types=[pltpu.VMEM((4, M), jnp.float32),
                       pltpu.VMEM((4, M // 128), jnp.float32),
                       pltpu.VMEM((M,), jnp.int32),
                       pltpu.VMEM((4, 16), jnp.int32),
                       pltpu.VMEM((4, 16), jnp.int32)],
        interpret=interpret,
    )(_sc_topk_body)
    idx, pv = sc_topk(sim, bm, memory_vid)

    nblk = 8
    blk = M // nblk
    total, local = pl.pallas_call(
        _k3_body,
        grid=(nblk, P),
        in_specs=[
            pl.BlockSpec((1, B, d), lambda m, p: (p, 0, 0)),
            pl.BlockSpec((1, blk, d), lambda m, p: (p, m, 0)),
            pl.BlockSpec((B, 16), lambda m, p: (0, 0)),
            pl.BlockSpec((B, 1), lambda m, p: (0, 0)),
            pl.BlockSpec((1, 1), lambda m, p: (0, 0)),
        ],
        out_specs=[
            pl.BlockSpec((1, 1), lambda m, p: (0, 0)),
            pl.BlockSpec((1, 1), lambda m, p: (0, 0)),
        ],
        out_shape=[jax.ShapeDtypeStruct((1, 1), jnp.float32),
                   jax.ShapeDtypeStruct((1, 1), jnp.float32)],
        scratch_shapes=[pltpu.VMEM((1, P * B), jnp.float32),
                        pltpu.VMEM((1, P * B), jnp.float32),
                        pltpu.VMEM((B, blk), jnp.float32),
                        pltpu.VMEM((1, blk), jnp.float32)],
        interpret=interpret,
    )(local_img_feats, centers, idx, pos2d, align)

    return (total[0, 0], local[0, 0], align[0, 0], pv[:, :_KTOP])


def kernel(global_img_feat, global_text_feat, local_img_feats,
           local_text_feats, centers, text_centers, memory_feats,
           position, memory_vid):
    return _impl(global_img_feat, global_text_feat, local_img_feats,
                 local_text_feats, centers, text_centers, memory_feats,
                 position, memory_vid)


# speedup vs baseline: 1.1119x; 1.1091x over previous
import functools

import jax
import jax.numpy as jnp
from jax import lax
from jax.experimental import pallas as pl
from jax.experimental.pallas import tpu as pltpu
from jax.experimental.pallas import tpu_sc as plsc

_SCALE = 10.0
_KTOP = 10
_TEMP = 0.5
_NEG_INF = -1e30
_GONE = -3.0e38


def _l2n(x):
    n = jnp.sqrt(jnp.sum(x * x, axis=-1, keepdims=True))
    return x / jnp.maximum(n, 1e-12)


def _bdot(a, b):
    return lax.dot_general(a.astype(jnp.bfloat16), b.astype(jnp.bfloat16),
                           (((1,), (1,)), ((), ())),
                           preferred_element_type=jnp.float32)


def _align_body(gif_ref, gtf_ref, align_ref):
    B = gif_ref.shape[0]
    img = _l2n(gif_ref[...])
    txt = _l2n(gtf_ref[...])
    diag = (lax.broadcasted_iota(jnp.int32, (B, B), 0)
            == lax.broadcasted_iota(jnp.int32, (B, B), 1))

    def sim_logits(x):
        return jnp.where(diag, _NEG_INF, _bdot(x, x) / _TEMP)

    img_sim = sim_logits(img)
    txt_sim = sim_logits(txt)

    def log_softmax(s):
        z = s - jnp.max(s, axis=1, keepdims=True)
        return z - jnp.log(jnp.sum(jnp.exp(z), axis=1, keepdims=True))

    img_logp = log_softmax(img_sim)
    txt_logp = log_softmax(txt_sim)
    img_p = jnp.exp(img_logp)
    txt_p = jnp.exp(txt_logp)

    def kl(logp, p):
        return jnp.sum(p * (jnp.log(jnp.maximum(p, 1e-12)) - logp)) / B

    align_ref[...] = jnp.reshape(
        0.5 * (kl(img_logp, txt_p) + kl(txt_logp, img_p)), (1, 1))


def _k1_body(ltf_ref, mem_ref, sim_ref, bm_ref, tm_ref):
    m = pl.program_id(0)

    @pl.when(m == 0)
    def _():
        tm_ref[...] = _l2n(jnp.mean(ltf_ref[...], axis=0))

    s = _bdot(tm_ref[...], _l2n(mem_ref[...]))
    sim_ref[...] = s
    B, mb1 = s.shape
    bm_ref[...] = jnp.max(s.reshape(B, mb1 // 128, 128), axis=2)[None]


def _sc_topk_body(sim_hbm, bm_hbm, vid_hbm, idx_hbm, pv_hbm,
                  simv, bmv, vidv, idxv, pvv):
    nc = 2
    wid = lax.axis_index("s") * nc + lax.axis_index("c")
    base = wid * 4
    pltpu.sync_copy(sim_hbm.at[pl.ds(base, 4)], simv)
    pltpu.sync_copy(bm_hbm.at[pl.ds(base, 4)], bmv)
    pltpu.sync_copy(vid_hbm, vidv)

    lane = lax.broadcasted_iota(jnp.int32, (16,), 0)

    idxvecs = [jnp.zeros((16,), jnp.int32) for _ in range(4)]
    for k in range(_KTOP):
        for r in range(4):
            b0 = bmv[r, pl.ds(0, 16)]
            b1 = bmv[r, pl.ds(16, 16)]
            b2 = bmv[r, pl.ds(32, 16)]
            b3 = bmv[r, pl.ds(48, 16)]
            gmax = jnp.max(jnp.maximum(jnp.maximum(b0, b1),
                                       jnp.maximum(b2, b3)))
            cpos = jnp.full((16,), 64, jnp.int32)
            for i, bi in enumerate((b0, b1, b2, b3)):
                cpos = jnp.minimum(cpos,
                                   jnp.where(bi == gmax, lane + 16 * i, 64))
            cstar = jnp.min(cpos)
            cbase = cstar * 128
            tpos = jnp.full((16,), 128, jnp.int32)
            for j in range(8):
                vj = simv[r, pl.ds(cbase + 16 * j, 16)]
                tpos = jnp.minimum(tpos,
                                   jnp.where(vj == gmax, lane + 16 * j, 128))
            mstar = cbase + jnp.min(tpos)
            idxvecs[r] = jnp.where(lane == k, mstar, idxvecs[r])
            plsc.store_scatter(simv,
                               [jnp.full((16,), r, jnp.int32),
                                jnp.full((16,), mstar, jnp.int32)],
                               jnp.full((16,), _GONE, jnp.float32),
                               mask=lane == 0)
            newmax = jnp.full((16,), _GONE, jnp.float32)
            for j in range(8):
                newmax = jnp.maximum(newmax,
                                     simv[r, pl.ds(cbase + 16 * j, 16)])
            plsc.store_scatter(bmv,
                               [jnp.full((16,), r, jnp.int32),
                                jnp.full((16,), cstar, jnp.int32)],
                               jnp.full((16,), jnp.max(newmax), jnp.float32),
                               mask=lane == 0)
    for r in range(4):
        idxv[r, :] = idxvecs[r]
        pvv[r, :] = plsc.load_gather(vidv, [idxvecs[r]])

    pltpu.sync_copy(idxv, idx_hbm.at[pl.ds(base, 4)])
    pltpu.sync_copy(pvv, pv_hbm.at[pl.ds(base, 4)])


def _k3_body(lif_ref, cen_ref, idx_ref, pos_ref, align_ref,
             total_ref, local_ref, spos_ref, sneg_ref, h_ref, msk_ref):
    mb = pl.program_id(0)
    p = pl.program_id(1)
    num_mb = pl.num_programs(0)
    num_p = pl.num_programs(1)

    pf = lif_ref[0]
    cb = cen_ref[0]
    B = pf.shape[0]
    blk = cb.shape[0]

    @pl.when(jnp.logical_and(mb == 0, p == 0))
    def _():
        spos_ref[...] = jnp.zeros_like(spos_ref)
        sneg_ref[...] = jnp.zeros_like(sneg_ref)

    @pl.when(p == 0)
    def _():
        base = mb * blk
        col = base + lax.broadcasted_iota(jnp.int32, (B, blk), 1)
        h = jnp.zeros((B, blk), jnp.float32)
        for k in range(_KTOP):
            h = h + (col == idx_ref[:, k:k + 1]).astype(jnp.float32)
        h_ref[...] = h
        hit = (jnp.sum(h, axis=0, keepdims=True)
               + jnp.sum((col == pos_ref[...]).astype(jnp.float32), axis=0,
                         keepdims=True))
        msk_ref[...] = jnp.where(hit > 0.0, 0.0, 1.0)

    a2 = jnp.sum(pf * pf, axis=1, keepdims=True)
    c2 = jnp.sum(cb * cb, axis=1, keepdims=True)
    d2 = a2 + c2.T - 2.0 * _bdot(pf, cb)
    dist = jnp.sqrt(jnp.maximum(d2, 1e-12))
    e = jnp.exp(-_SCALE * dist)

    lanes = pl.ds(p * B, B)
    spos_ref[:, lanes] += jnp.sum(e * h_ref[...], axis=1, keepdims=True).T
    sneg_ref[:, lanes] += jnp.sum(e * msk_ref[...], axis=1, keepdims=True).T

    @pl.when(jnp.logical_and(mb == num_mb - 1, p == num_p - 1))
    def _():
        acc = jnp.zeros((1, 1), jnp.float32)
        for q in range(4):
            x = jnp.log(spos_ref[:, q * B:(q + 1) * B])
            y = jnp.log(sneg_ref[:, q * B:(q + 1) * B])
            l = jnp.sum(y - x) / B
            l = jnp.where(jnp.isnan(l), 0.0, l)
            acc = acc + jnp.reshape(l, (1, 1))
        loc = acc / num_p
        local_ref[...] = loc
        total_ref[...] = loc + 0.5 * align_ref[...]


def _impl(global_img_feat, global_text_feat, local_img_feats,
          local_text_feats, centers, text_centers, memory_feats,
          position, memory_vid, interpret=False):
    B, d = global_img_feat.shape
    P, M, _ = centers.shape
    pos2d = position.reshape(B, 1)

    nb1 = 8
    mb1 = M // nb1
    sim, bm = pl.pallas_call(
        _k1_body,
        grid=(nb1,),
        in_specs=[
            pl.BlockSpec((P, B, d), lambda m: (0, 0, 0)),
            pl.BlockSpec((mb1, d), lambda m: (m, 0)),
        ],
        out_specs=[
            pl.BlockSpec((B, mb1), lambda m: (0, m)),
            pl.BlockSpec((1, B, mb1 // 128), lambda m: (m, 0, 0)),
        ],
        out_shape=[jax.ShapeDtypeStruct((B, M), jnp.float32),
                   jax.ShapeDtypeStruct((nb1, B, mb1 // 128), jnp.float32)],
        scratch_shapes=[pltpu.VMEM((B, d), jnp.float32)],
        interpret=interpret,
    )(local_text_feats, memory_feats)
    bm = bm.transpose(1, 0, 2).reshape(B, M // 128)

    align = pl.pallas_call(
        _align_body,
        out_shape=jax.ShapeDtypeStruct((1, 1), jnp.float32),
        interpret=interpret,
    )(global_img_feat, global_text_feat)

    sc_topk = functools.partial(
        pl.kernel,
        out_type=[jax.ShapeDtypeStruct((B, 16), jnp.int32),
                  jax.ShapeDtypeStruct((B, 16), jnp.int32)],
        mesh=plsc.VectorSubcoreMesh(core_axis_name="c", subcore_axis_name="s"),
        compiler_params=pltpu.CompilerParams(needs_layout_passes=False),
        scratch_types=[pltpu.VMEM((4, M), jnp.float32),
                       pltpu.VMEM((4, M // 128), jnp.float32),
                       pltpu.VMEM((M,), jnp.int32),
                       pltpu.VMEM((4, 16), jnp.int32),
                       pltpu.VMEM((4, 16), jnp.int32)],
        interpret=interpret,
    )(_sc_topk_body)
    idx, pv = sc_topk(sim, bm, memory_vid)

    nblk = 4
    blk = M // nblk
    total, local = pl.pallas_call(
        _k3_body,
        grid=(nblk, P),
        in_specs=[
            pl.BlockSpec((1, B, d), lambda m, p: (p, 0, 0)),
            pl.BlockSpec((1, blk, d), lambda m, p: (p, m, 0)),
            pl.BlockSpec((B, 16), lambda m, p: (0, 0)),
            pl.BlockSpec((B, 1), lambda m, p: (0, 0)),
            pl.BlockSpec((1, 1), lambda m, p: (0, 0)),
        ],
        out_specs=[
            pl.BlockSpec((1, 1), lambda m, p: (0, 0)),
            pl.BlockSpec((1, 1), lambda m, p: (0, 0)),
        ],
        out_shape=[jax.ShapeDtypeStruct((1, 1), jnp.float32),
                   jax.ShapeDtypeStruct((1, 1), jnp.float32)],
        scratch_shapes=[pltpu.VMEM((1, P * B), jnp.float32),
                        pltpu.VMEM((1, P * B), jnp.float32),
                        pltpu.VMEM((B, blk), jnp.float32),
                        pltpu.VMEM((1, blk), jnp.float32)],
        interpret=interpret,
    )(local_img_feats, centers, idx, pos2d, align)

    return (total[0, 0], local[0, 0], align[0, 0], pv[:, :_KTOP])


def kernel(global_img_feat, global_text_feat, local_img_feats,
           local_text_feats, centers, text_centers, memory_feats,
           position, memory_vid):
    return _impl(global_img_feat, global_text_feat, local_img_feats,
                 local_text_feats, centers, text_centers, memory_feats,
                 position, memory_vid)
